# baseline (speedup 1.0000x reference)
import jax
import jax.numpy as jnp
from jax.experimental import pallas as pl
from jax.experimental.pallas import tpu as pltpu


def _gcn_fused_kernel(x_ref, m_ref, b1_ref, w2_ref, b2_ref, o_ref):
    h = jnp.dot(x_ref[...], m_ref[...], preferred_element_type=jnp.float32)
    h = jnp.maximum(h + b1_ref[...], 0.0).astype(jnp.bfloat16)
    out = jnp.dot(h, w2_ref[...], preferred_element_type=jnp.float32)
    o_ref[...] = out + b2_ref[...]


@jax.jit
def kernel(a_norm, x, w1, b1, w2t, b2):
    B, N, f_in = x.shape
    f_hid = w1.shape[1]
    y_dim = w2t.shape[1]
    nf = N * f_hid

    m = (a_norm.T[:, :, None] * w1[0][None, None, :]).reshape(N, nf)
    m = m.astype(jnp.bfloat16)
    b1_t = jnp.tile(b1, (1, N))
    x_rows = x[..., 0].astype(jnp.bfloat16)
    w2_b = w2t.astype(jnp.bfloat16)

    blk_b = 128
    grid = (B // blk_b,)

    out = pl.pallas_call(
        _gcn_fused_kernel,
        out_shape=jax.ShapeDtypeStruct((B, y_dim), jnp.float32),
        grid=grid,
        in_specs=[
            pl.BlockSpec((blk_b, N), lambda i: (i, 0)),
            pl.BlockSpec((N, nf), lambda i: (0, 0)),
            pl.BlockSpec((1, nf), lambda i: (0, 0)),
            pl.BlockSpec((nf, y_dim), lambda i: (0, 0)),
            pl.BlockSpec((1, y_dim), lambda i: (0, 0)),
        ],
        out_specs=pl.BlockSpec((blk_b, y_dim), lambda i: (i, 0)),
        compiler_params=pltpu.CompilerParams(
            dimension_semantics=("parallel",),
        ),
    )(x_rows, m, b1_t, w2_b, b2)

    return out


# speedup vs baseline: 1.5965x; 1.5965x over previous
import functools

import jax
import jax.numpy as jnp
from jax.experimental import pallas as pl
from jax.experimental.pallas import tpu as pltpu


def _gcn_kernel(x_ref, at_ref, w1t_ref, b1t_ref, w2_ref, o_ref,
                s_ref, *, ck, nk, f_hid):
    c = pl.program_id(0)
    k = pl.program_id(1)

    @pl.when(k == 0)
    def _init():
        s_ref[...] = jnp.dot(x_ref[...], at_ref[...],
                             preferred_element_type=jnp.float32)

    kg = c * nk + k
    n_iota = jax.lax.broadcasted_iota(jnp.int32, (at_ref.shape[1], ck), 0)
    j_iota = jax.lax.broadcasted_iota(jnp.int32, (at_ref.shape[1], ck), 1)
    node_of_j = (kg * ck + j_iota) // f_hid
    r_k = (node_of_j == n_iota).astype(jnp.float32)

    h = jnp.dot(s_ref[...], r_k, preferred_element_type=jnp.float32)
    h = jnp.maximum(h * w1t_ref[...] + b1t_ref[...], 0.0)

    contrib = jnp.dot(h, w2_ref[...], preferred_element_type=jnp.float32)

    @pl.when(k == 0)
    def _first():
        o_ref[...] = contrib[None]

    @pl.when(k != 0)
    def _rest():
        o_ref[...] += contrib[None]


@jax.jit
def kernel(a_norm, x, w1, b1, w2t, b2):
    B, N, f_in = x.shape
    f_hid = w1.shape[1]
    y_dim = w2t.shape[1]
    nf = N * f_hid

    a_t = a_norm.T
    w1_t = jnp.tile(w1, (1, N))
    b1_t = jnp.tile(b1, (1, N))
    x_rows = x[..., 0]

    n_cores = 2
    ck = 1024
    nk = nf // (n_cores * ck)

    partials = pl.pallas_call(
        functools.partial(_gcn_kernel, ck=ck, nk=nk, f_hid=f_hid),
        out_shape=jax.ShapeDtypeStruct((n_cores, B, y_dim), jnp.float32),
        grid=(n_cores, nk),
        in_specs=[
            pl.BlockSpec((B, N), lambda c, k: (0, 0)),
            pl.BlockSpec((N, N), lambda c, k: (0, 0)),
            pl.BlockSpec((1, ck), lambda c, k: (0, c * nk + k)),
            pl.BlockSpec((1, ck), lambda c, k: (0, c * nk + k)),
            pl.BlockSpec((ck, y_dim), lambda c, k: (c * nk + k, 0)),
        ],
        out_specs=pl.BlockSpec((1, B, y_dim), lambda c, k: (c, 0, 0)),
        scratch_shapes=[pltpu.VMEM((B, N), jnp.float32)],
        compiler_params=pltpu.CompilerParams(
            dimension_semantics=("parallel", "arbitrary"),
        ),
    )(x_rows, a_t, w1_t, b1_t, w2t)

    return partials[0] + partials[1] + b2


# speedup vs baseline: 2.0882x; 1.3080x over previous
import functools

import jax
import jax.numpy as jnp
from jax.experimental import pallas as pl
from jax.experimental.pallas import tpu as pltpu

_NSTREAM = 2


def _gcn_kernel(x_ref, at_ref, b2_ref, *stream_refs, o_ref, s_ref,
                ck, f_hid):
    k = pl.program_id(0)
    n = at_ref.shape[1]

    @pl.when(k == 0)
    def _init():
        s_ref[...] = jnp.dot(x_ref[...], at_ref[...],
                             preferred_element_type=jnp.float32)
        o_ref[...] = jnp.broadcast_to(b2_ref[...], o_ref.shape)

    n_iota = jax.lax.broadcasted_iota(jnp.int32, (n, ck), 0)
    j_node = jax.lax.broadcasted_iota(jnp.int32, (n, ck), 1) // f_hid

    acc = jnp.zeros_like(o_ref)
    for t in range(_NSTREAM):
        w1t_ref = stream_refs[3 * t]
        b1t_ref = stream_refs[3 * t + 1]
        w2_ref = stream_refs[3 * t + 2]
        kg = k * _NSTREAM + t
        node_of_j = kg * (ck // f_hid) + j_node
        r_w = jnp.where(node_of_j == n_iota, w1t_ref[...], 0.0)
        h = jnp.dot(s_ref[...], r_w, preferred_element_type=jnp.float32)
        h = jnp.maximum(h + b1t_ref[...], 0.0)
        acc = acc + jnp.dot(h, w2_ref[...],
                            preferred_element_type=jnp.float32)
    o_ref[...] += acc


@jax.jit
def kernel(a_norm, x, w1, b1, w2t, b2):
    B, N, f_in = x.shape
    f_hid = w1.shape[1]
    y_dim = w2t.shape[1]
    nf = N * f_hid

    a_t = a_norm.T
    w1_t = jnp.tile(w1, (1, N))
    b1_t = jnp.tile(b1, (1, N))
    x_rows = x[..., 0]

    ck = 1024
    nk = nf // (_NSTREAM * ck)

    stream_specs = []
    stream_args = []
    for t in range(_NSTREAM):
        stream_specs += [
            pl.BlockSpec((1, ck), functools.partial(
                lambda t, k: (0, _NSTREAM * k + t), t)),
            pl.BlockSpec((1, ck), functools.partial(
                lambda t, k: (0, _NSTREAM * k + t), t)),
            pl.BlockSpec((ck, y_dim), functools.partial(
                lambda t, k: (_NSTREAM * k + t, 0), t)),
        ]
        stream_args += [w1_t, b1_t, w2t]

    def body(x_ref, at_ref, b2_ref, *rest):
        *stream_refs, o_ref, s_ref = rest
        _gcn_kernel(x_ref, at_ref, b2_ref, *stream_refs,
                    o_ref=o_ref, s_ref=s_ref, ck=ck, f_hid=f_hid)

    out = pl.pallas_call(
        body,
        out_shape=jax.ShapeDtypeStruct((B, y_dim), jnp.float32),
        grid=(nk,),
        in_specs=[
            pl.BlockSpec((B, N), lambda k: (0, 0)),
            pl.BlockSpec((N, N), lambda k: (0, 0)),
            pl.BlockSpec((1, y_dim), lambda k: (0, 0)),
        ] + stream_specs,
        out_specs=pl.BlockSpec((B, y_dim), lambda k: (0, 0)),
        scratch_shapes=[pltpu.VMEM((B, N), jnp.float32)],
        compiler_params=pltpu.CompilerParams(
            dimension_semantics=("arbitrary",),
        ),
    )(x_rows, a_t, b2, *stream_args)

    return out
